# baseline (speedup 1.0000x reference)
import functools

import jax
import jax.numpy as jnp
from jax.experimental import pallas as pl
from jax.experimental.pallas import tpu as pltpu

_FAST = jax.lax.Precision.DEFAULT


def _pass1_body(x_ref, adj_ref, w1_ref, b1_ref, w2_ref,
                s2_ref, s1_ref, *, block):
    i = pl.program_id(0)

    @pl.when(i == 0)
    def _():
        s1_ref[...] = jnp.dot(x_ref[...], w1_ref[...], precision=_FAST,
                              preferred_element_type=jnp.float32)

    h = jnp.dot(adj_ref[...], s1_ref[...], precision=_FAST,
                preferred_element_type=jnp.float32)
    h = jnp.maximum(h + b1_ref[...], 0.0)
    s2_ref[...] = jnp.dot(h, w2_ref[...], precision=_FAST,
                          preferred_element_type=jnp.float32)


def _pass2_body(adj_ref, s2_ref, b2_ref, out_ref):
    out_ref[...] = (
        jnp.dot(adj_ref[...], s2_ref[...], precision=_FAST,
                preferred_element_type=jnp.float32)
        + b2_ref[...])


def kernel(x, adj, W1, b1, W2, b2):
    n, nfeat = x.shape
    hid1 = W1.shape[1]
    nout = W2.shape[1]

    block1 = next(b for b in (400, 200, 100, 50, 25, 20, 10, 8, 5, 4, 2, 1)
                  if n % b == 0)
    block2 = block1

    b1r = b1.reshape(1, hid1)
    b2r = b2.reshape(1, nout)

    s2 = pl.pallas_call(
        functools.partial(_pass1_body, block=block1),
        grid=(n // block1,),
        in_specs=[
            pl.BlockSpec((n, nfeat), lambda i: (0, 0)),
            pl.BlockSpec((block1, n), lambda i: (i, 0)),
            pl.BlockSpec((nfeat, hid1), lambda i: (0, 0)),
            pl.BlockSpec((1, hid1), lambda i: (0, 0)),
            pl.BlockSpec((hid1, nout), lambda i: (0, 0)),
        ],
        out_specs=pl.BlockSpec((block1, nout), lambda i: (i, 0)),
        out_shape=jax.ShapeDtypeStruct((n, nout), jnp.float32),
        scratch_shapes=[pltpu.VMEM((n, hid1), jnp.float32)],
        compiler_params=pltpu.CompilerParams(
            dimension_semantics=("arbitrary",),
        ),
    )(x, adj, W1, b1r, W2)

    out = pl.pallas_call(
        _pass2_body,
        grid=(n // block2,),
        in_specs=[
            pl.BlockSpec((block2, n), lambda i: (i, 0)),
            pl.BlockSpec((n, nout), lambda i: (0, 0)),
            pl.BlockSpec((1, nout), lambda i: (0, 0)),
        ],
        out_specs=pl.BlockSpec((block2, nout), lambda i: (i, 0)),
        out_shape=jax.ShapeDtypeStruct((n, nout), jnp.float32),
        compiler_params=pltpu.CompilerParams(
            dimension_semantics=("arbitrary",),
        ),
    )(adj, s2, b2r)
    return out


# speedup vs baseline: 1.0249x; 1.0249x over previous
import functools

import jax
import jax.numpy as jnp
from jax.experimental import pallas as pl
from jax.experimental.pallas import tpu as pltpu

_FAST = jax.lax.Precision.DEFAULT


def _gcn_body(x_ref, adj_ref, w1_ref, b1_ref, w2_ref, b2_ref,
              out_ref, s1_ref, s2_ref, *, block):
    p = pl.program_id(0)
    i = pl.program_id(1)

    @pl.when(jnp.logical_and(p == 0, i == 0))
    def _():
        s1_ref[...] = jnp.dot(x_ref[...], w1_ref[...], precision=_FAST,
                              preferred_element_type=jnp.float32)

    @pl.when(p == 0)
    def _():
        h = jnp.dot(adj_ref[...], s1_ref[...], precision=_FAST,
                    preferred_element_type=jnp.float32)
        h = jnp.maximum(h + b1_ref[...], 0.0)
        s2_ref[pl.ds(i * block, block), :] = jnp.dot(
            h, w2_ref[...], precision=_FAST,
            preferred_element_type=jnp.float32)

    @pl.when(p == 1)
    def _():
        out_ref[...] = (
            jnp.dot(adj_ref[...], s2_ref[...], precision=_FAST,
                    preferred_element_type=jnp.float32)
            + b2_ref[...])


def kernel(x, adj, W1, b1, W2, b2):
    n, nfeat = x.shape
    hid1 = W1.shape[1]
    nout = W2.shape[1]

    block = next(b for b in (400, 200, 100, 50, 25, 20, 10, 8, 5, 4, 2, 1)
                 if n % b == 0)
    grid = (2, n // block)

    b1r = b1.reshape(1, hid1)
    b2r = b2.reshape(1, nout)

    out = pl.pallas_call(
        functools.partial(_gcn_body, block=block),
        grid=grid,
        in_specs=[
            pl.BlockSpec((n, nfeat), lambda p, i: (0, 0)),
            pl.BlockSpec((block, n), lambda p, i: (i, 0)),
            pl.BlockSpec((nfeat, hid1), lambda p, i: (0, 0)),
            pl.BlockSpec((1, hid1), lambda p, i: (0, 0)),
            pl.BlockSpec((hid1, nout), lambda p, i: (0, 0)),
            pl.BlockSpec((1, nout), lambda p, i: (0, 0)),
        ],
        out_specs=pl.BlockSpec((block, nout), lambda p, i: (i, 0)),
        out_shape=jax.ShapeDtypeStruct((n, nout), jnp.float32),
        scratch_shapes=[
            pltpu.VMEM((n, hid1), jnp.float32),
            pltpu.VMEM((n, nout), jnp.float32),
        ],
        compiler_params=pltpu.CompilerParams(
            dimension_semantics=("arbitrary", "arbitrary"),
        ),
    )(x, adj, W1, b1r, W2, b2r)
    return out
